# Initial kernel scaffold; baseline (speedup 1.0000x reference)
import jax
import jax.numpy as jnp
from jax.experimental import pallas as pl


def kernel(x, table):
    raise NotImplementedError("write your pallas kernel here")


# speedup vs baseline: 1.4954x; 1.4954x over previous
import functools

import jax
import jax.numpy as jnp
from jax import lax
from jax.experimental import pallas as pl
from jax.experimental.pallas import tpu as pltpu
from jax.experimental.pallas import tpu_sc as plsc

NC = 2
NS = 16
NW = NC * NS

IDX_PER_STREAM = 128
STREAMS_PER_CHUNK = 20
CHUNK = IDX_PER_STREAM * STREAMS_PER_CHUNK


def _gather_body(n_per_w, n_chunks, x_hbm, table_hbm, out_hbm,
                 idx_v, rows_v, gsem, wsem):
  wid = lax.axis_index("s") * NC + lax.axis_index("c")
  base = wid * n_per_w
  pltpu.sync_copy(x_hbm.at[wid], idx_v)

  def chunk_body(g, carry):
    copies = []
    for j in range(STREAMS_PER_CHUNK):
      copies.append(pltpu.async_copy(
          table_hbm.at[idx_v.at[g * STREAMS_PER_CHUNK + j]],
          rows_v.at[pl.ds(j * IDX_PER_STREAM, IDX_PER_STREAM)],
          gsem))
    for c in copies:
      c.wait()
    pltpu.sync_copy(rows_v, out_hbm.at[pl.ds(base + g * CHUNK, CHUNK)])
    return carry

  lax.fori_loop(0, n_chunks, chunk_body, 0)


def kernel(x, table):
  B, H = x.shape
  V, D = table.shape
  N = B * H
  assert N % (NW * CHUNK) == 0
  n_per_w = N // NW
  n_chunks = n_per_w // CHUNK

  x_flat = x.reshape(NW, n_per_w // IDX_PER_STREAM, IDX_PER_STREAM)

  mesh = plsc.VectorSubcoreMesh(core_axis_name="c", subcore_axis_name="s")
  grid_kernel = pl.kernel(
      functools.partial(_gather_body, n_per_w, n_chunks),
      out_type=jax.ShapeDtypeStruct((N, D), jnp.float32),
      mesh=mesh,
      scratch_types=[
          pltpu.VMEM((n_per_w // IDX_PER_STREAM, IDX_PER_STREAM), jnp.int32),
          pltpu.VMEM((CHUNK, D), jnp.float32),
          pltpu.SemaphoreType.DMA,
          pltpu.SemaphoreType.DMA,
      ],
      compiler_params=pltpu.CompilerParams(use_tc_tiling_on_sc=False),
  )
  out = grid_kernel(x_flat, table)
  return out.reshape(B, H, D)
